# baseline (speedup 1.0000x reference)
import functools

import jax
import jax.numpy as jnp
from jax import lax
from jax.experimental import pallas as pl
from jax.experimental.pallas import tpu as pltpu
from jax.experimental.pallas import tpu_sc as plsc

NUM_FIELDS = 26
VOCAB = 100000
EMBED_DIM = 32
BATCH = 16384

_NC = 2
_NS = 16
_NW = _NC * _NS

_TOTAL = BATCH * NUM_FIELDS
_GATHER = 128
_NGATH = 13
_CHUNK = _GATHER * _NGATH
_NCHUNKS = _TOTAL // _CHUNK
_CHUNKS_PER_W = _NCHUNKS // _NW
_PERIOD = 208


def _sc_gather_kernel(idx_hbm, tab_hbm, out_hbm, patt_v, idx_v, rows_v, sem):
    wid = lax.axis_index("s") * _NC + lax.axis_index("c")

    lanes = lax.iota(jnp.int32, 16)
    for k in range(_PERIOD // 16):
        q = lanes + (16 * k)
        patt_v[pl.ds(16 * k, 16)] = (q % NUM_FIELDS) * VOCAB

    def body(c, carry):
        chunk_id = wid * _CHUNKS_PER_W + c
        pltpu.sync_copy(idx_hbm.at[chunk_id], idx_v)
        for j in range(_NGATH):
            for i in range(_GATHER // 16):
                off = (j * _GATHER + i * 16) % _PERIOD
                idx_v[j, pl.ds(i * 16, 16)] = (
                    idx_v[j, pl.ds(i * 16, 16)] + patt_v[pl.ds(off, 16)]
                )
        descs = []
        for j in range(_NGATH):
            descs.append(
                pltpu.async_copy(
                    tab_hbm.at[idx_v.at[j]],
                    rows_v.at[pl.ds(j * _GATHER, _GATHER)],
                    sem,
                )
            )
        for d in descs:
            d.wait()
        pltpu.sync_copy(rows_v, out_hbm.at[chunk_id])
        return carry

    lax.fori_loop(0, _CHUNKS_PER_W, body, 0)


@jax.jit
def kernel(category_inputs, tables):
    idx3 = category_inputs.reshape(_NCHUNKS, _NGATH, _GATHER)
    tab2 = tables.reshape(NUM_FIELDS * VOCAB, EMBED_DIM)

    mesh = plsc.VectorSubcoreMesh(core_axis_name="c", subcore_axis_name="s")
    run = functools.partial(
        pl.kernel,
        out_type=jax.ShapeDtypeStruct((_NCHUNKS, _CHUNK, EMBED_DIM), jnp.float32),
        mesh=mesh,
        scratch_types=[
            pltpu.VMEM((_PERIOD,), jnp.int32),
            pltpu.VMEM((_NGATH, _GATHER), jnp.int32),
            pltpu.VMEM((_CHUNK, EMBED_DIM), jnp.float32),
            pltpu.SemaphoreType.DMA,
        ],
        compiler_params=pltpu.CompilerParams(use_tc_tiling_on_sc=False),
    )(_sc_gather_kernel)
    out = run(idx3, tab2)
    return out.reshape(BATCH, NUM_FIELDS * EMBED_DIM)


# speedup vs baseline: 1.0013x; 1.0013x over previous
import functools

import jax
import jax.numpy as jnp
from jax import lax
from jax.experimental import pallas as pl
from jax.experimental.pallas import tpu as pltpu
from jax.experimental.pallas import tpu_sc as plsc

NUM_FIELDS = 26
VOCAB = 100000
EMBED_DIM = 32
BATCH = 16384

_NC = 2
_NS = 16
_NW = _NC * _NS

_TOTAL = BATCH * NUM_FIELDS
_GATHER = 128
_NGATH = 13
_CHUNK = _GATHER * _NGATH
_NCHUNKS = _TOTAL // _CHUNK
_CHUNKS_PER_W = _NCHUNKS // _NW
_PERIOD = 208


def _sc_gather_kernel(
    idx_hbm, tab_hbm, out_hbm,
    patt_v, idx0, idx1, rows0, rows1,
    isem0, isem1, gsem, wsem0, wsem1,
):
    wid = lax.axis_index("s") * _NC + lax.axis_index("c")
    base = wid * _CHUNKS_PER_W

    lanes = lax.iota(jnp.int32, 16)
    for k in range(_PERIOD // 16):
        q = lanes + (16 * k)
        patt_v[pl.ds(16 * k, 16)] = (q % NUM_FIELDS) * VOCAB

    pltpu.async_copy(idx_hbm.at[base], idx0, isem0)
    pltpu.async_copy(idx_hbm.at[base + 1], idx1, isem1)

    def chunk_step(c, idx_v, rows_v, isem, wsem):
        chunk_id = base + c
        pltpu.make_async_copy(idx_hbm.at[base], idx_v, isem).wait()
        for j in range(_NGATH):
            for i in range(_GATHER // 16):
                off = (j * _GATHER + i * 16) % _PERIOD
                idx_v[j, pl.ds(i * 16, 16)] = (
                    idx_v[j, pl.ds(i * 16, 16)] + patt_v[pl.ds(off, 16)]
                )
        @pl.when(c >= 2)
        def _():
            pltpu.make_async_copy(out_hbm.at[base], rows_v, wsem).wait()

        descs = []
        for j in range(_NGATH):
            descs.append(
                pltpu.async_copy(
                    tab_hbm.at[idx_v.at[j]],
                    rows_v.at[pl.ds(j * _GATHER, _GATHER)],
                    gsem,
                )
            )
        for d in descs:
            d.wait()
        pltpu.async_copy(rows_v, out_hbm.at[chunk_id], wsem)
        @pl.when(c + 2 < _CHUNKS_PER_W)
        def _():
            pltpu.async_copy(idx_hbm.at[chunk_id + 2], idx_v, isem)

    def body(c, carry):
        @pl.when(c % 2 == 0)
        def _():
            chunk_step(c, idx0, rows0, isem0, wsem0)

        @pl.when(c % 2 == 1)
        def _():
            chunk_step(c, idx1, rows1, isem1, wsem1)

        return carry

    lax.fori_loop(0, _CHUNKS_PER_W, body, 0)

    pltpu.make_async_copy(out_hbm.at[base], rows0, wsem0).wait()
    pltpu.make_async_copy(out_hbm.at[base], rows1, wsem1).wait()


@jax.jit
def kernel(category_inputs, tables):
    idx3 = category_inputs.reshape(_NCHUNKS, _NGATH, _GATHER)
    tab2 = tables.reshape(NUM_FIELDS * VOCAB, EMBED_DIM)

    mesh = plsc.VectorSubcoreMesh(core_axis_name="c", subcore_axis_name="s")
    run = functools.partial(
        pl.kernel,
        out_type=jax.ShapeDtypeStruct((_NCHUNKS, _CHUNK, EMBED_DIM), jnp.float32),
        mesh=mesh,
        scratch_types=[
            pltpu.VMEM((_PERIOD,), jnp.int32),
            pltpu.VMEM((_NGATH, _GATHER), jnp.int32),
            pltpu.VMEM((_NGATH, _GATHER), jnp.int32),
            pltpu.VMEM((_CHUNK, EMBED_DIM), jnp.float32),
            pltpu.VMEM((_CHUNK, EMBED_DIM), jnp.float32),
            pltpu.SemaphoreType.DMA,
            pltpu.SemaphoreType.DMA,
            pltpu.SemaphoreType.DMA,
            pltpu.SemaphoreType.DMA,
            pltpu.SemaphoreType.DMA,
        ],
        compiler_params=pltpu.CompilerParams(use_tc_tiling_on_sc=False),
    )(_sc_gather_kernel)
    out = run(idx3, tab2)
    return out.reshape(BATCH, NUM_FIELDS * EMBED_DIM)


# speedup vs baseline: 1.0041x; 1.0028x over previous
import functools

import jax
import jax.numpy as jnp
from jax import lax
from jax.experimental import pallas as pl
from jax.experimental.pallas import tpu as pltpu
from jax.experimental.pallas import tpu_sc as plsc

NUM_FIELDS = 26
VOCAB = 100000
EMBED_DIM = 32
BATCH = 16384

_NC = 2
_NS = 16
_NW = _NC * _NS

_TOTAL = BATCH * NUM_FIELDS
_GATHER = 128
_NGATH = 13
_CHUNK = _GATHER * _NGATH
_NCHUNKS = _TOTAL // _CHUNK
_CHUNKS_PER_W = _NCHUNKS // _NW
_PERIOD = 208


def _sc_gather_kernel(
    idx_hbm, tab_hbm, out_hbm,
    patt_v, idx0, idx1, rows0, rows1,
    isem0, isem1, gsem0, gsem1, wsem0, wsem1,
):
    wid = lax.axis_index("s") * _NC + lax.axis_index("c")
    base = wid * _CHUNKS_PER_W

    pltpu.async_copy(idx_hbm.at[base], idx0, isem0)

    lanes = lax.iota(jnp.int32, 16)
    for k in range(_PERIOD // 16):
        q = lanes + (16 * k)
        patt_v[pl.ds(16 * k, 16)] = (q % NUM_FIELDS) * VOCAB

    def body(c, carry):
        @pl.when(c % 2 == 0)
        def _():
            step_even(c)

        @pl.when(c % 2 == 1)
        def _():
            step_odd(c)

        return carry

    def step_even(c):
        chunk_id = base + c
        pltpu.make_async_copy(idx_hbm.at[base], idx0, isem0).wait()
        for j in range(_NGATH):
            for i in range(_GATHER // 16):
                off = (j * _GATHER + i * 16) % _PERIOD
                idx0[j, pl.ds(i * 16, 16)] = (
                    idx0[j, pl.ds(i * 16, 16)] + patt_v[pl.ds(off, 16)]
                )
        @pl.when(c >= 2)
        def _():
            pltpu.make_async_copy(out_hbm.at[base], rows0, wsem0).wait()

        for j in range(_NGATH):
            pltpu.async_copy(
                tab_hbm.at[idx0.at[j]],
                rows0.at[pl.ds(j * _GATHER, _GATHER)],
                gsem0,
            )

        @pl.when(c >= 1)
        def _():
            for j in range(_NGATH):
                pltpu.make_async_copy(
                    tab_hbm.at[idx1.at[j]],
                    rows1.at[pl.ds(j * _GATHER, _GATHER)],
                    gsem1,
                ).wait()
            pltpu.async_copy(rows1, out_hbm.at[chunk_id - 1], wsem1)

        @pl.when(c + 1 < _CHUNKS_PER_W)
        def _():
            pltpu.async_copy(idx_hbm.at[chunk_id + 1], idx1, isem1)

    def step_odd(c):
        chunk_id = base + c
        pltpu.make_async_copy(idx_hbm.at[base], idx1, isem1).wait()
        for j in range(_NGATH):
            for i in range(_GATHER // 16):
                off = (j * _GATHER + i * 16) % _PERIOD
                idx1[j, pl.ds(i * 16, 16)] = (
                    idx1[j, pl.ds(i * 16, 16)] + patt_v[pl.ds(off, 16)]
                )
        @pl.when(c >= 2)
        def _():
            pltpu.make_async_copy(out_hbm.at[base], rows1, wsem1).wait()

        for j in range(_NGATH):
            pltpu.async_copy(
                tab_hbm.at[idx1.at[j]],
                rows1.at[pl.ds(j * _GATHER, _GATHER)],
                gsem1,
            )

        for j in range(_NGATH):
            pltpu.make_async_copy(
                tab_hbm.at[idx0.at[j]],
                rows0.at[pl.ds(j * _GATHER, _GATHER)],
                gsem0,
            ).wait()
        pltpu.async_copy(rows0, out_hbm.at[chunk_id - 1], wsem0)

        @pl.when(c + 1 < _CHUNKS_PER_W)
        def _():
            pltpu.async_copy(idx_hbm.at[chunk_id + 1], idx0, isem0)

    lax.fori_loop(0, _CHUNKS_PER_W, body, 0)

    for j in range(_NGATH):
        pltpu.make_async_copy(
            tab_hbm.at[idx1.at[j]],
            rows1.at[pl.ds(j * _GATHER, _GATHER)],
            gsem1,
        ).wait()
    pltpu.async_copy(rows1, out_hbm.at[base + _CHUNKS_PER_W - 1], wsem1)
    pltpu.make_async_copy(out_hbm.at[base], rows0, wsem0).wait()
    pltpu.make_async_copy(out_hbm.at[base], rows1, wsem1).wait()


@jax.jit
def kernel(category_inputs, tables):
    idx3 = category_inputs.reshape(_NCHUNKS, _NGATH, _GATHER)
    tab2 = tables.reshape(NUM_FIELDS * VOCAB, EMBED_DIM)

    mesh = plsc.VectorSubcoreMesh(core_axis_name="c", subcore_axis_name="s")
    run = functools.partial(
        pl.kernel,
        out_type=jax.ShapeDtypeStruct((_NCHUNKS, _CHUNK, EMBED_DIM), jnp.float32),
        mesh=mesh,
        scratch_types=[
            pltpu.VMEM((_PERIOD,), jnp.int32),
            pltpu.VMEM((_NGATH, _GATHER), jnp.int32),
            pltpu.VMEM((_NGATH, _GATHER), jnp.int32),
            pltpu.VMEM((_CHUNK, EMBED_DIM), jnp.float32),
            pltpu.VMEM((_CHUNK, EMBED_DIM), jnp.float32),
            pltpu.SemaphoreType.DMA,
            pltpu.SemaphoreType.DMA,
            pltpu.SemaphoreType.DMA,
            pltpu.SemaphoreType.DMA,
            pltpu.SemaphoreType.DMA,
            pltpu.SemaphoreType.DMA,
        ],
        compiler_params=pltpu.CompilerParams(use_tc_tiling_on_sc=False),
    )(_sc_gather_kernel)
    out = run(idx3, tab2)
    return out.reshape(BATCH, NUM_FIELDS * EMBED_DIM)
